# baseline (speedup 1.0000x reference)
import functools

import jax
import jax.numpy as jnp
from jax import lax
from jax.experimental import pallas as pl
from jax.experimental.pallas import tpu as pltpu
from jax.experimental.pallas import tpu_sc as plsc

BATCH = 16384
CAT_DIM = 256
USER_DIM = 512
NC = 2
NS = 16
NW = NC * NS
ROWS_PER_W = BATCH // NW
CHUNK = 64
NCHUNK = ROWS_PER_W // CHUNK
L = 16
KT = CAT_DIM // L
NGRP = CHUNK // L
HG = 8


def _body(cat_ids, user_ids, lat, lon, wd, bd, emb_cat, emb_user,
          out,
          idxc, idxu, lat_v, lon_v, out_v,
          cat0, cat1, usr0, usr1, w_v, b_v,
          semc0, semc1, semu0, semu1, sem_st):
    wid = lax.axis_index("s") * NC + lax.axis_index("c")
    base = wid * ROWS_PER_W
    stage = [
        pltpu.make_async_copy(wd, w_v, sem_st),
        pltpu.make_async_copy(bd, b_v, sem_st),
        pltpu.make_async_copy(cat_ids.at[pl.ds(base, ROWS_PER_W)], idxc, sem_st),
        pltpu.make_async_copy(user_ids.at[pl.ds(base, ROWS_PER_W)], idxu, sem_st),
        pltpu.make_async_copy(lat.at[pl.ds(base, ROWS_PER_W)], lat_v, sem_st),
        pltpu.make_async_copy(lon.at[pl.ds(base, ROWS_PER_W)], lon_v, sem_st),
    ]
    for cp in stage:
        cp.start()
    for cp in stage:
        cp.wait()
    lanes = lax.iota(jnp.int32, L)

    catb = [cat0, cat1]
    usrb = [usr0, usr1]
    semc = [semc0, semc1]
    semu = [semu0, semu1]

    def issue(c, b):
        pltpu.async_copy(emb_cat.at[idxc.at[pl.ds(c * CHUNK, CHUNK)]],
                         catb[b], semc[b])
        pltpu.async_copy(emb_user.at[idxu.at[pl.ds(c * CHUNK, CHUNK)]],
                         usrb[b], semu[b])

    def wait(c, b):
        pltpu.make_async_copy(emb_cat.at[idxc.at[pl.ds(c * CHUNK, CHUNK)]],
                              catb[b], semc[b]).wait()
        pltpu.make_async_copy(emb_user.at[idxu.at[pl.ds(c * CHUNK, CHUNK)]],
                              usrb[b], semu[b]).wait()

    def compute(c, b):
        cr = catb[b]
        ur = usrb[b]

        def gbody(g, _g):
            vbase = g * L
            obase = c * CHUNK + vbase
            lat16 = lat_v[pl.ds(obase, L)]
            lon16 = lon_v[pl.ds(obase, L)]
            z = jnp.zeros((L,), jnp.float32)
            res = z
            for h in range(2):
                hoff = h * HG

                def kbody(k, accs):
                    kf = k * L
                    w0v = w_v[0, pl.ds(kf, L)]
                    w1v = w_v[1, pl.ds(kf, L)]
                    bv = b_v[pl.ds(kf, L)]
                    acc, s0, s1 = accs
                    acc, s0, s1 = list(acc), list(s0), list(s1)
                    for r in range(HG):
                        catv = cr[vbase + hoff + r, pl.ds(kf, L)]
                        u1v = ur[vbase + hoff + r, pl.ds(kf, L)]
                        u2v = ur[vbase + hoff + r, pl.ds(kf + CAT_DIM, L)]
                        acc[r] = acc[r] + catv * u1v + bv * u2v
                        s0[r] = s0[r] + w0v * u2v
                        s1[r] = s1[r] + w1v * u2v
                    return tuple(acc), tuple(s0), tuple(s1)

                init = ((z,) * HG,) * 3
                acc, s0, s1 = lax.fori_loop(0, KT, kbody, init)
                for r in range(HG):
                    t = acc[r] + lat16[hoff + r] * s0[r] + lon16[hoff + r] * s1[r]
                    res = jnp.where(lanes == hoff + r, jnp.sum(t), res)
            out_v[pl.ds(obase, L)] = res
            return _g

        lax.fori_loop(0, NGRP, gbody, 0)

    issue(0, 0)

    def tbody(t, _t):
        c0 = 2 * t
        issue(c0 + 1, 1)
        wait(c0, 0)
        compute(c0, 0)

        @pl.when(t < NCHUNK // 2 - 1)
        def _prefetch():
            issue(c0 + 2, 0)

        wait(c0 + 1, 1)
        compute(c0 + 1, 1)
        return _t

    lax.fori_loop(0, NCHUNK // 2, tbody, 0)
    pltpu.sync_copy(out_v, out.at[pl.ds(base, ROWS_PER_W)])


def kernel(category_ids, poi_lat, poi_lon, user_ids, W_dense, b_dense, emb_cat, emb_user):
    cat_ids = category_ids.reshape(BATCH).astype(jnp.int32)
    uids = user_ids.reshape(BATCH).astype(jnp.int32)
    lat = poi_lat.reshape(BATCH)
    lon = poi_lon.reshape(BATCH)
    mesh = plsc.VectorSubcoreMesh(core_axis_name="c", subcore_axis_name="s")
    f = pl.kernel(
        _body,
        out_type=jax.ShapeDtypeStruct((BATCH,), jnp.float32),
        mesh=mesh,
        compiler_params=pltpu.CompilerParams(needs_layout_passes=False),
        scratch_types=[
            pltpu.VMEM((ROWS_PER_W,), jnp.int32),
            pltpu.VMEM((ROWS_PER_W,), jnp.int32),
            pltpu.VMEM((ROWS_PER_W,), jnp.float32),
            pltpu.VMEM((ROWS_PER_W,), jnp.float32),
            pltpu.VMEM((ROWS_PER_W,), jnp.float32),
            pltpu.VMEM((CHUNK, CAT_DIM), jnp.float32),
            pltpu.VMEM((CHUNK, CAT_DIM), jnp.float32),
            pltpu.VMEM((CHUNK, USER_DIM), jnp.float32),
            pltpu.VMEM((CHUNK, USER_DIM), jnp.float32),
            pltpu.VMEM((2, CAT_DIM), jnp.float32),
            pltpu.VMEM((CAT_DIM,), jnp.float32),
            pltpu.SemaphoreType.DMA,
            pltpu.SemaphoreType.DMA,
            pltpu.SemaphoreType.DMA,
            pltpu.SemaphoreType.DMA,
            pltpu.SemaphoreType.DMA,
        ],
    )
    dot = f(cat_ids, uids, lat, lon, W_dense, b_dense, emb_cat, emb_user)
    return dot.reshape(BATCH, 1, 1)


# speedup vs baseline: 1.0181x; 1.0181x over previous
import functools

import jax
import jax.numpy as jnp
from jax import lax
from jax.experimental import pallas as pl
from jax.experimental.pallas import tpu as pltpu
from jax.experimental.pallas import tpu_sc as plsc

BATCH = 16384
CAT_DIM = 256
USER_DIM = 512
NC = 2
NS = 16
NW = NC * NS
ROWS_PER_W = BATCH // NW
CHUNK = 32
NCHUNK = ROWS_PER_W // CHUNK
L = 16
KT = CAT_DIM // L
NGRP = CHUNK // L
HG = 8


def _body(cat_ids, user_ids, lat, lon, wd, bd, emb_cat, emb_user,
          out,
          idxc, idxu, lat_v, lon_v, out_v,
          cat0, cat1, usr0, usr1, w_v, b_v,
          semc0, semc1, semu0, semu1, sem_st):
    wid = lax.axis_index("s") * NC + lax.axis_index("c")
    base = wid * ROWS_PER_W
    stage = [
        pltpu.make_async_copy(wd, w_v, sem_st),
        pltpu.make_async_copy(bd, b_v, sem_st),
        pltpu.make_async_copy(cat_ids.at[pl.ds(base, ROWS_PER_W)], idxc, sem_st),
        pltpu.make_async_copy(user_ids.at[pl.ds(base, ROWS_PER_W)], idxu, sem_st),
        pltpu.make_async_copy(lat.at[pl.ds(base, ROWS_PER_W)], lat_v, sem_st),
        pltpu.make_async_copy(lon.at[pl.ds(base, ROWS_PER_W)], lon_v, sem_st),
    ]
    for cp in stage:
        cp.start()
    for cp in stage:
        cp.wait()
    lanes = lax.iota(jnp.int32, L)

    catb = [cat0, cat1]
    usrb = [usr0, usr1]
    semc = [semc0, semc1]
    semu = [semu0, semu1]

    def issue(c, b):
        pltpu.async_copy(emb_cat.at[idxc.at[pl.ds(c * CHUNK, CHUNK)]],
                         catb[b], semc[b])
        pltpu.async_copy(emb_user.at[idxu.at[pl.ds(c * CHUNK, CHUNK)]],
                         usrb[b], semu[b])

    def wait(c, b):
        pltpu.make_async_copy(emb_cat.at[idxc.at[pl.ds(c * CHUNK, CHUNK)]],
                              catb[b], semc[b]).wait()
        pltpu.make_async_copy(emb_user.at[idxu.at[pl.ds(c * CHUNK, CHUNK)]],
                              usrb[b], semu[b]).wait()

    def compute(c, b):
        cr = catb[b]
        ur = usrb[b]

        def gbody(g, _g):
            vbase = g * L
            obase = c * CHUNK + vbase
            lat16 = lat_v[pl.ds(obase, L)]
            lon16 = lon_v[pl.ds(obase, L)]
            z = jnp.zeros((L,), jnp.float32)
            res = z
            for h in range(2):
                hoff = h * HG

                def kbody(k, accs):
                    kf = k * L
                    w0v = w_v[0, pl.ds(kf, L)]
                    w1v = w_v[1, pl.ds(kf, L)]
                    bv = b_v[pl.ds(kf, L)]
                    acc, s0, s1 = accs
                    acc, s0, s1 = list(acc), list(s0), list(s1)
                    for r in range(HG):
                        catv = cr[vbase + hoff + r, pl.ds(kf, L)]
                        u1v = ur[vbase + hoff + r, pl.ds(kf, L)]
                        u2v = ur[vbase + hoff + r, pl.ds(kf + CAT_DIM, L)]
                        acc[r] = acc[r] + catv * u1v + bv * u2v
                        s0[r] = s0[r] + w0v * u2v
                        s1[r] = s1[r] + w1v * u2v
                    return tuple(acc), tuple(s0), tuple(s1)

                init = ((z,) * HG,) * 3
                acc, s0, s1 = lax.fori_loop(0, KT, kbody, init)
                for r in range(HG):
                    t = acc[r] + lat16[hoff + r] * s0[r] + lon16[hoff + r] * s1[r]
                    res = jnp.where(lanes == hoff + r, jnp.sum(t), res)
            out_v[pl.ds(obase, L)] = res
            return _g

        lax.fori_loop(0, NGRP, gbody, 0)

    issue(0, 0)

    def tbody(t, _t):
        c0 = 2 * t
        issue(c0 + 1, 1)
        wait(c0, 0)
        compute(c0, 0)

        @pl.when(t < NCHUNK // 2 - 1)
        def _prefetch():
            issue(c0 + 2, 0)

        wait(c0 + 1, 1)
        compute(c0 + 1, 1)
        return _t

    lax.fori_loop(0, NCHUNK // 2, tbody, 0)
    pltpu.sync_copy(out_v, out.at[pl.ds(base, ROWS_PER_W)])


def kernel(category_ids, poi_lat, poi_lon, user_ids, W_dense, b_dense, emb_cat, emb_user):
    cat_ids = category_ids.reshape(BATCH).astype(jnp.int32)
    uids = user_ids.reshape(BATCH).astype(jnp.int32)
    lat = poi_lat.reshape(BATCH)
    lon = poi_lon.reshape(BATCH)
    mesh = plsc.VectorSubcoreMesh(core_axis_name="c", subcore_axis_name="s")
    f = pl.kernel(
        _body,
        out_type=jax.ShapeDtypeStruct((BATCH,), jnp.float32),
        mesh=mesh,
        compiler_params=pltpu.CompilerParams(needs_layout_passes=False),
        scratch_types=[
            pltpu.VMEM((ROWS_PER_W,), jnp.int32),
            pltpu.VMEM((ROWS_PER_W,), jnp.int32),
            pltpu.VMEM((ROWS_PER_W,), jnp.float32),
            pltpu.VMEM((ROWS_PER_W,), jnp.float32),
            pltpu.VMEM((ROWS_PER_W,), jnp.float32),
            pltpu.VMEM((CHUNK, CAT_DIM), jnp.float32),
            pltpu.VMEM((CHUNK, CAT_DIM), jnp.float32),
            pltpu.VMEM((CHUNK, USER_DIM), jnp.float32),
            pltpu.VMEM((CHUNK, USER_DIM), jnp.float32),
            pltpu.VMEM((2, CAT_DIM), jnp.float32),
            pltpu.VMEM((CAT_DIM,), jnp.float32),
            pltpu.SemaphoreType.DMA,
            pltpu.SemaphoreType.DMA,
            pltpu.SemaphoreType.DMA,
            pltpu.SemaphoreType.DMA,
            pltpu.SemaphoreType.DMA,
        ],
    )
    dot = f(cat_ids, uids, lat, lon, W_dense, b_dense, emb_cat, emb_user)
    return dot.reshape(BATCH, 1, 1)
